# Initial kernel scaffold; baseline (speedup 1.0000x reference)
import jax
import jax.numpy as jnp
from jax.experimental import pallas as pl


def kernel(x, W1, b1, W2, b2):
    raise NotImplementedError("write your pallas kernel here")


# speedup vs baseline: 5.2962x; 5.2962x over previous
import functools

import jax
import jax.numpy as jnp
from jax import lax
from jax.experimental import pallas as pl
from jax.experimental.pallas import tpu as pltpu
from jax.experimental.pallas import tpu_sc as plsc

_K = 16
_IMAX = jnp.iinfo(jnp.int32).max
_NEG = float("-inf")


def _leaky(v):
    return jnp.where(v >= 0, v, 0.01 * v)


def _mlp_body(x_ref, w1t_ref, b1_ref, w2t_ref, b2_ref, h_ref, hn_ref):
    x = x_ref[...]
    h1 = lax.dot_general(
        x.astype(jnp.bfloat16), w1t_ref[...].astype(jnp.bfloat16),
        (((1,), (0,)), ((), ())), preferred_element_type=jnp.float32)
    h1 = _leaky(h1 + b1_ref[...])
    h2 = lax.dot_general(
        h1.astype(jnp.bfloat16), w2t_ref[...].astype(jnp.bfloat16),
        (((1,), (0,)), ((), ())), preferred_element_type=jnp.float32)
    h = _leaky(h2 + b2_ref[...])
    nrm = jnp.maximum(jnp.sqrt(jnp.sum(h * h, axis=1, keepdims=True)), 1e-12)
    h_ref[...] = h
    hn_ref[...] = h / nrm


def _mlp_forward(x, w1t, b1, w2t, b2, block_rows):
    n, d = x.shape
    dh = w1t.shape[1]
    grid = n // block_rows
    return pl.pallas_call(
        _mlp_body,
        grid=(grid,),
        in_specs=[
            pl.BlockSpec((block_rows, d), lambda i: (i, 0)),
            pl.BlockSpec((d, dh), lambda i: (0, 0)),
            pl.BlockSpec((1, dh), lambda i: (0, 0)),
            pl.BlockSpec((dh, d), lambda i: (0, 0)),
            pl.BlockSpec((1, d), lambda i: (0, 0)),
        ],
        out_specs=[
            pl.BlockSpec((block_rows, d), lambda i: (i, 0)),
            pl.BlockSpec((block_rows, d), lambda i: (i, 0)),
        ],
        out_shape=[
            jax.ShapeDtypeStruct((n, d), jnp.float32),
            jax.ShapeDtypeStruct((n, d), jnp.float32),
        ],
    )(x, w1t, b1, w2t, b2)


def _topk_body(hn16_ref, qt_ref, idx_ref, *, n_keys, bk, bq):
    nkb = n_keys // bk
    qt = qt_ref[...]

    def merge_block(j, carry):
        rv, ri = carry
        kb = hn16_ref[pl.ds(j * bk, bk), :]
        s = lax.dot_general(kb, qt, (((1,), (0,)), ((), ())),
                            preferred_element_type=jnp.float32)
        ci = lax.broadcasted_iota(jnp.int32, (bk, bq), 0) + j * bk
        c = jnp.concatenate([rv, s], axis=0)
        cidx = jnp.concatenate([ri, ci], axis=0)
        vals, ids = [], []
        for _ in range(_K):
            m = jnp.max(c, axis=0, keepdims=True)
            idc = jnp.where(c == m, cidx, _IMAX)
            chosen = jnp.min(idc, axis=0, keepdims=True)
            vals.append(m)
            ids.append(chosen)
            c = jnp.where(cidx == chosen, _NEG, c)
        return jnp.concatenate(vals, axis=0), jnp.concatenate(ids, axis=0)

    rv0 = jnp.full((_K, bq), _NEG, jnp.float32)
    ri0 = jnp.full((_K, bq), _IMAX, jnp.int32)
    _, ri = lax.fori_loop(0, nkb, merge_block, (rv0, ri0))
    idx_ref[...] = ri


def _topk_indices(hn16, hnt16, bk, bq):
    n_keys, d = hn16.shape
    np_ = hnt16.shape[1]
    body = functools.partial(_topk_body, n_keys=n_keys, bk=bk, bq=bq)
    return pl.pallas_call(
        body,
        grid=(np_ // bq,),
        in_specs=[
            pl.BlockSpec((n_keys, d), lambda i: (0, 0)),
            pl.BlockSpec((d, bq), lambda i: (0, i)),
        ],
        out_specs=pl.BlockSpec((_K, bq), lambda i: (0, i)),
        out_shape=jax.ShapeDtypeStruct((_K, np_), jnp.int32),
    )(hn16, hnt16)


def _edge_attr_sc(h, flat_idx, cq):
    n, d = h.shape
    np_ = flat_idx.shape[0] // _K
    nw = 32
    qw = np_ // nw
    nch = qw // cq
    lanes = 16
    nl = d // lanes

    mesh = plsc.VectorSubcoreMesh(
        core_axis_name="c", subcore_axis_name="s",
        num_cores=2, num_subcores=16)

    @functools.partial(
        pl.kernel,
        out_type=jax.ShapeDtypeStruct((np_, d), jnp.float32),
        mesh=mesh,
        scratch_types=[
            pltpu.VMEM((cq * _K,), jnp.int32),
            pltpu.VMEM((cq * _K, d), jnp.float32),
            pltpu.VMEM((cq, d), jnp.float32),
            pltpu.SemaphoreType.DMA,
        ],
    )
    def edge_attr_kernel(h_hbm, idx_hbm, out_hbm, idx_v, rows_v, acc_v, sem):
        wid = lax.axis_index("s") * 2 + lax.axis_index("c")

        def chunk(ch, carry):
            qbase = wid * qw + ch * cq
            pltpu.sync_copy(idx_hbm.at[pl.ds(qbase * _K, cq * _K)], idx_v)
            pltpu.async_copy(h_hbm.at[idx_v], rows_v, sem).wait()
            for q in range(cq):
                for l in range(nl):
                    a = rows_v[q * _K, pl.ds(l * lanes, lanes)]
                    for r in range(1, _K):
                        a = a + rows_v[q * _K + r, pl.ds(l * lanes, lanes)]
                    acc_v[q, pl.ds(l * lanes, lanes)] = a * 0.0625
            pltpu.sync_copy(acc_v, out_hbm.at[pl.ds(qbase, cq)])
            return carry

        lax.fori_loop(0, nch, chunk, 0)

    return edge_attr_kernel(h, flat_idx)


def kernel(x, W1, b1, W2, b2):
    n, d = x.shape
    dh = W1.shape[0]
    bq = 256
    bk = 2000
    np_ = ((n + bq - 1) // bq) * bq

    h, hn = _mlp_forward(
        x, W1.T, b1.reshape(1, dh), W2.T, b2.reshape(1, d), block_rows=1000)

    hn16 = hn.astype(jnp.bfloat16)
    hnt16 = jnp.pad(hn16, ((0, np_ - n), (0, 0))).T
    idx_t = _topk_indices(hn16, hnt16, bk=bk, bq=bq)

    idx = idx_t[:, :n].T
    nbr = idx.reshape(-1)
    qry = jnp.repeat(jnp.arange(n, dtype=nbr.dtype), _K)
    edge_index = jnp.stack([nbr, qry], axis=0)

    flat_idx = jnp.clip(idx_t.T, 0, n - 1).reshape(-1)
    edge_attr = _edge_attr_sc(h, flat_idx, cq=8)[:n]

    return (h, edge_index, edge_attr)


# speedup vs baseline: 5.7972x; 1.0946x over previous
import functools

import jax
import jax.numpy as jnp
from jax import lax
from jax.experimental import pallas as pl
from jax.experimental.pallas import tpu as pltpu
from jax.experimental.pallas import tpu_sc as plsc

_K = 16
_IMAX = jnp.iinfo(jnp.int32).max
_NEG = float("-inf")


def _leaky(v):
    return jnp.where(v >= 0, v, 0.01 * v)


def _mlp_body(x_ref, w1t_ref, b1_ref, w2t_ref, b2_ref, h_ref, hn_ref):
    x = x_ref[...]
    h1 = lax.dot_general(
        x.astype(jnp.bfloat16), w1t_ref[...].astype(jnp.bfloat16),
        (((1,), (0,)), ((), ())), preferred_element_type=jnp.float32)
    h1 = _leaky(h1 + b1_ref[...])
    h2 = lax.dot_general(
        h1.astype(jnp.bfloat16), w2t_ref[...].astype(jnp.bfloat16),
        (((1,), (0,)), ((), ())), preferred_element_type=jnp.float32)
    h = _leaky(h2 + b2_ref[...])
    nrm = jnp.maximum(jnp.sqrt(jnp.sum(h * h, axis=1, keepdims=True)), 1e-12)
    h_ref[...] = h
    hn_ref[...] = h / nrm


def _mlp_forward(x, w1t, b1, w2t, b2, block_rows):
    n, d = x.shape
    dh = w1t.shape[1]
    grid = n // block_rows
    return pl.pallas_call(
        _mlp_body,
        grid=(grid,),
        in_specs=[
            pl.BlockSpec((block_rows, d), lambda i: (i, 0)),
            pl.BlockSpec((d, dh), lambda i: (0, 0)),
            pl.BlockSpec((1, dh), lambda i: (0, 0)),
            pl.BlockSpec((dh, d), lambda i: (0, 0)),
            pl.BlockSpec((1, d), lambda i: (0, 0)),
        ],
        out_specs=[
            pl.BlockSpec((block_rows, d), lambda i: (i, 0)),
            pl.BlockSpec((block_rows, d), lambda i: (i, 0)),
        ],
        out_shape=[
            jax.ShapeDtypeStruct((n, d), jnp.float32),
            jax.ShapeDtypeStruct((n, d), jnp.float32),
        ],
    )(x, w1t, b1, w2t, b2)


def _topk_body(hn16_ref, qt_ref, idx_ref, *, n_keys, bk, bq):
    nkb = n_keys // bk
    qt = qt_ref[...]

    def merge_block(j, carry):
        rv, ri = carry
        kb = hn16_ref[pl.ds(j * bk, bk), :]
        s = lax.dot_general(kb, qt, (((1,), (0,)), ((), ())),
                            preferred_element_type=jnp.float32)
        ci = lax.broadcasted_iota(jnp.int32, (bk, bq), 0) + j * bk
        c = jnp.concatenate([rv, s], axis=0)
        cidx = jnp.concatenate([ri, ci], axis=0)
        vals, ids = [], []
        for _ in range(_K):
            m = jnp.max(c, axis=0, keepdims=True)
            idc = jnp.where(c == m, cidx, _IMAX)
            chosen = jnp.min(idc, axis=0, keepdims=True)
            vals.append(m)
            ids.append(chosen)
            c = jnp.where(cidx == chosen, _NEG, c)
        return jnp.concatenate(vals, axis=0), jnp.concatenate(ids, axis=0)

    rv0 = jnp.full((_K, bq), _NEG, jnp.float32)
    ri0 = jnp.full((_K, bq), _IMAX, jnp.int32)
    _, ri = lax.fori_loop(0, nkb, merge_block, (rv0, ri0))
    idx_ref[...] = ri


def _topk_indices(hn16, hnt16, bk, bq):
    n_keys, d = hn16.shape
    np_ = hnt16.shape[1]
    body = functools.partial(_topk_body, n_keys=n_keys, bk=bk, bq=bq)
    return pl.pallas_call(
        body,
        grid=(np_ // bq,),
        in_specs=[
            pl.BlockSpec((n_keys, d), lambda i: (0, 0)),
            pl.BlockSpec((d, bq), lambda i: (0, i)),
        ],
        out_specs=pl.BlockSpec((_K, bq), lambda i: (0, i)),
        out_shape=jax.ShapeDtypeStruct((_K, np_), jnp.int32),
    )(hn16, hnt16)


def _edge_attr_sc(h, flat_idx, cq):
    n, d = h.shape
    np_ = flat_idx.shape[0] // _K
    nw = 32
    qw = np_ // nw
    nch = qw // cq
    lanes = 16
    nl = d // lanes

    mesh = plsc.VectorSubcoreMesh(
        core_axis_name="c", subcore_axis_name="s",
        num_cores=2, num_subcores=16)

    @functools.partial(
        pl.kernel,
        out_type=jax.ShapeDtypeStruct((np_, d), jnp.float32),
        mesh=mesh,
        scratch_types=[
            pltpu.VMEM((cq * _K,), jnp.int32),
            pltpu.VMEM((cq * _K, d), jnp.float32),
            pltpu.VMEM((cq, d), jnp.float32),
            pltpu.SemaphoreType.DMA,
        ],
    )
    def edge_attr_kernel(h_hbm, idx_hbm, out_hbm, idx_v, rows_v, acc_v, sem):
        wid = lax.axis_index("s") * 2 + lax.axis_index("c")

        def chunk(ch, carry):
            qbase = wid * qw + ch * cq
            pltpu.sync_copy(idx_hbm.at[pl.ds(qbase * _K, cq * _K)], idx_v)
            pltpu.async_copy(h_hbm.at[idx_v], rows_v, sem).wait()
            for q in range(cq):
                for l in range(nl):
                    a = rows_v[q * _K, pl.ds(l * lanes, lanes)]
                    for r in range(1, _K):
                        a = a + rows_v[q * _K + r, pl.ds(l * lanes, lanes)]
                    acc_v[q, pl.ds(l * lanes, lanes)] = a * 0.0625
            pltpu.sync_copy(acc_v, out_hbm.at[pl.ds(qbase, cq)])
            return carry

        lax.fori_loop(0, nch, chunk, 0)

    return edge_attr_kernel(h, flat_idx)


def kernel(x, W1, b1, W2, b2):
    n, d = x.shape
    dh = W1.shape[0]
    bq = 512
    bk = 2000
    np_ = ((n + bq - 1) // bq) * bq

    h, hn = _mlp_forward(
        x, W1.T, b1.reshape(1, dh), W2.T, b2.reshape(1, d), block_rows=1000)

    hn16 = hn.astype(jnp.bfloat16)
    hnt16 = jnp.pad(hn16, ((0, np_ - n), (0, 0))).T
    idx_t = _topk_indices(hn16, hnt16, bk=bk, bq=bq)

    idx = idx_t[:, :n].T
    nbr = idx.reshape(-1)
    qry = jnp.repeat(jnp.arange(n, dtype=nbr.dtype), _K)
    edge_index = jnp.stack([nbr, qry], axis=0)

    flat_idx = jnp.clip(idx_t.T, 0, n - 1).reshape(-1)
    edge_attr = _edge_attr_sc(h, flat_idx, cq=8)[:n]

    return (h, edge_index, edge_attr)


# speedup vs baseline: 6.4238x; 1.1081x over previous
import functools

import jax
import jax.numpy as jnp
from jax import lax
from jax.experimental import pallas as pl
from jax.experimental.pallas import tpu as pltpu
from jax.experimental.pallas import tpu_sc as plsc

_K = 16
_IMAX = jnp.iinfo(jnp.int32).max
_NEG = float("-inf")


def _leaky(v):
    return jnp.where(v >= 0, v, 0.01 * v)


def _mlp_body(x_ref, w1t_ref, b1_ref, w2t_ref, b2_ref, h_ref, hn_ref):
    x = x_ref[...]
    h1 = lax.dot_general(
        x.astype(jnp.bfloat16), w1t_ref[...].astype(jnp.bfloat16),
        (((1,), (0,)), ((), ())), preferred_element_type=jnp.float32)
    h1 = _leaky(h1 + b1_ref[...])
    h2 = lax.dot_general(
        h1.astype(jnp.bfloat16), w2t_ref[...].astype(jnp.bfloat16),
        (((1,), (0,)), ((), ())), preferred_element_type=jnp.float32)
    h = _leaky(h2 + b2_ref[...])
    nrm = jnp.maximum(jnp.sqrt(jnp.sum(h * h, axis=1, keepdims=True)), 1e-12)
    h_ref[...] = h
    hn_ref[...] = h / nrm


def _mlp_forward(x, w1t, b1, w2t, b2, block_rows):
    n, d = x.shape
    dh = w1t.shape[1]
    grid = n // block_rows
    return pl.pallas_call(
        _mlp_body,
        grid=(grid,),
        in_specs=[
            pl.BlockSpec((block_rows, d), lambda i: (i, 0)),
            pl.BlockSpec((d, dh), lambda i: (0, 0)),
            pl.BlockSpec((1, dh), lambda i: (0, 0)),
            pl.BlockSpec((dh, d), lambda i: (0, 0)),
            pl.BlockSpec((1, d), lambda i: (0, 0)),
        ],
        out_specs=[
            pl.BlockSpec((block_rows, d), lambda i: (i, 0)),
            pl.BlockSpec((block_rows, d), lambda i: (i, 0)),
        ],
        out_shape=[
            jax.ShapeDtypeStruct((n, d), jnp.float32),
            jax.ShapeDtypeStruct((n, d), jnp.float32),
        ],
    )(x, w1t, b1, w2t, b2)


def _topk_body(hn16_ref, qt_ref, idx_ref, *, n_keys, bk, bq):
    nkb = n_keys // bk
    qt = qt_ref[...]

    def merge_block(j, carry):
        rv, ri = carry
        kb = hn16_ref[pl.ds(j * bk, bk), :]
        s = lax.dot_general(kb, qt, (((1,), (0,)), ((), ())),
                            preferred_element_type=jnp.float32)
        ci = lax.broadcasted_iota(jnp.int32, (bk, bq), 0) + j * bk
        c = jnp.concatenate([rv, s], axis=0)
        cidx = jnp.concatenate([ri, ci], axis=0)
        vals, ids = [], []
        for _ in range(_K):
            m = jnp.max(c, axis=0, keepdims=True)
            idc = jnp.where(c == m, cidx, _IMAX)
            chosen = jnp.min(idc, axis=0, keepdims=True)
            vals.append(m)
            ids.append(chosen)
            c = jnp.where(cidx == chosen, _NEG, c)
        return jnp.concatenate(vals, axis=0), jnp.concatenate(ids, axis=0)

    rv0 = jnp.full((_K, bq), _NEG, jnp.float32)
    ri0 = jnp.full((_K, bq), _IMAX, jnp.int32)
    _, ri = lax.fori_loop(0, nkb, merge_block, (rv0, ri0))
    idx_ref[...] = ri


def _topk_indices(hn16, hnt16, bk, bq):
    n_keys, d = hn16.shape
    np_ = hnt16.shape[1]
    body = functools.partial(_topk_body, n_keys=n_keys, bk=bk, bq=bq)
    return pl.pallas_call(
        body,
        grid=(np_ // bq,),
        in_specs=[
            pl.BlockSpec((n_keys, d), lambda i: (0, 0)),
            pl.BlockSpec((d, bq), lambda i: (0, i)),
        ],
        out_specs=pl.BlockSpec((_K, bq), lambda i: (0, i)),
        out_shape=jax.ShapeDtypeStruct((_K, np_), jnp.int32),
    )(hn16, hnt16)


def _edge_attr_sc(h, flat_idx, cq):
    n, d = h.shape
    np_ = flat_idx.shape[0] // _K
    nw = 32
    qw = np_ // nw
    nch = qw // cq
    lanes = 16
    nl = d // lanes

    mesh = plsc.VectorSubcoreMesh(
        core_axis_name="c", subcore_axis_name="s",
        num_cores=2, num_subcores=16)

    @functools.partial(
        pl.kernel,
        out_type=jax.ShapeDtypeStruct((np_, d), jnp.float32),
        mesh=mesh,
        scratch_types=[
            pltpu.VMEM((cq * _K,), jnp.int32),
            pltpu.VMEM((cq * _K, d), jnp.float32),
            pltpu.VMEM((cq, d), jnp.float32),
            pltpu.SemaphoreType.DMA,
        ],
    )
    def edge_attr_kernel(h_hbm, idx_hbm, out_hbm, idx_v, rows_v, acc_v, sem):
        wid = lax.axis_index("s") * 2 + lax.axis_index("c")

        def chunk(ch, carry):
            qbase = wid * qw + ch * cq
            pltpu.sync_copy(idx_hbm.at[pl.ds(qbase * _K, cq * _K)], idx_v)
            pltpu.async_copy(h_hbm.at[idx_v], rows_v, sem).wait()
            for q in range(cq):
                for l in range(nl):
                    a = rows_v[q * _K, pl.ds(l * lanes, lanes)]
                    for r in range(1, _K):
                        a = a + rows_v[q * _K + r, pl.ds(l * lanes, lanes)]
                    acc_v[q, pl.ds(l * lanes, lanes)] = a * 0.0625
            pltpu.sync_copy(acc_v, out_hbm.at[pl.ds(qbase, cq)])
            return carry

        lax.fori_loop(0, nch, chunk, 0)

    return edge_attr_kernel(h, flat_idx)


def kernel(x, W1, b1, W2, b2):
    n, d = x.shape
    dh = W1.shape[0]
    bq = 1024
    bk = 2000
    np_ = ((n + bq - 1) // bq) * bq

    h, hn = _mlp_forward(
        x, W1.T, b1.reshape(1, dh), W2.T, b2.reshape(1, d), block_rows=1000)

    hn16 = hn.astype(jnp.bfloat16)
    hnt16 = jnp.pad(hn16, ((0, np_ - n), (0, 0))).T
    idx_t = _topk_indices(hn16, hnt16, bk=bk, bq=bq)

    idx = idx_t[:, :n].T
    nbr = idx.reshape(-1)
    qry = jnp.repeat(jnp.arange(n, dtype=nbr.dtype), _K)
    edge_index = jnp.stack([nbr, qry], axis=0)

    flat_idx = jnp.clip(idx_t.T, 0, n - 1).reshape(-1)
    edge_attr = _edge_attr_sc(h, flat_idx, cq=8)[:n]

    return (h, edge_index, edge_attr)
